# baseline (speedup 1.0000x reference)
import functools

import jax
import jax.numpy as jnp
from jax.experimental import pallas as pl
from jax.experimental.pallas import tpu as pltpu
from jax.experimental.pallas import tpu_sc as plsc


def _pair_body(t_ref, out_ref):
    x = t_ref[...]
    w = x.shape[1]
    y = jnp.transpose(x)
    v = y.reshape(w // 2, 2, x.shape[0])
    out_ref[...] = jnp.concatenate([v[:, 0, :], v[:, 1, :]], axis=1)


def _tc_pair_table(tt):
    d, v = tt.shape
    w = 12800
    grid = (v + w - 1) // w
    return pl.pallas_call(
        _pair_body,
        grid=(grid,),
        in_specs=[pl.BlockSpec((d, w), lambda i: (0, i))],
        out_specs=pl.BlockSpec((w // 2, 2 * d), lambda i: (i, 0)),
        out_shape=jax.ShapeDtypeStruct((v // 2, 2 * d), jnp.float32),
    )(tt)


def _sc_gather(table2, idx):
    n = idx.shape[0]
    d = table2.shape[1]
    nc, ns = 2, 16
    nw = nc * ns
    b_per_w = n // nw
    ch = 800
    n_ch = b_per_w // ch
    mesh = plsc.VectorSubcoreMesh(core_axis_name="c", subcore_axis_name="s")

    @functools.partial(
        pl.kernel, mesh=mesh,
        out_type=jax.ShapeDtypeStruct((n, d), jnp.float32),
        scratch_types=[
            pltpu.VMEM((ch,), jnp.int32),
            pltpu.VMEM((ch, d), jnp.float32),
            pltpu.SemaphoreType.DMA,
        ],
    )
    def k(table_hbm, idx_hbm, out_hbm, idx_v, rows_v, sem):
        wid = jax.lax.axis_index("s") * nc + jax.lax.axis_index("c")
        base = wid * b_per_w

        @pl.loop(0, n_ch)
        def _(c):
            off = base + c * ch
            pltpu.sync_copy(idx_hbm.at[pl.ds(off, ch)], idx_v)
            pltpu.async_copy(table_hbm.at[idx_v], rows_v, sem).wait()
            pltpu.sync_copy(rows_v, out_hbm.at[pl.ds(off, ch)])

    return k(table2, idx)


def _fused_body(x_ref, p_ref, m_ref, s_ref, s0_ref, m1_ref,
                mpw_ref, wa_ref, va_ref, out_ref):
    bb, ll = p_ref.shape
    d = x_ref.shape[1] // 2
    f32 = jnp.float32

    bf16 = jnp.bfloat16
    xb = x_ref[...].astype(bf16)
    sg = s_ref[...]
    s0 = s0_ref[...]

    u = jax.lax.dot(sg, p_ref[...].astype(bf16),
                    preferred_element_type=f32).astype(bf16)
    pf = jax.lax.dot(u * m1_ref[...], jnp.ones((ll, 1), bf16),
                     preferred_element_type=f32)

    mpw = mpw_ref[...]
    wc = jax.lax.dot(mpw, wa_ref[...],
                     preferred_element_type=f32).astype(bf16)
    zeros = jnp.zeros((d, d), bf16)
    wcl = jnp.concatenate([wc, zeros], axis=0)
    wcr = jnp.concatenate([zeros, wc], axis=0)
    zl = jax.lax.dot(xb, wcl, preferred_element_type=f32)
    zr = jax.lax.dot(xb, wcr, preferred_element_type=f32)
    t = jnp.tanh(jnp.where(pf > 0.5, zr, zl))
    g1 = jax.lax.dot(t.astype(bf16), va_ref[...].astype(bf16),
                     preferred_element_type=f32)
    g = g1.reshape(bb, ll)

    mx = jnp.max(g, axis=1, keepdims=True)
    e = jnp.exp(g - mx)
    alpha = e / jnp.sum(e, axis=1, keepdims=True)

    em0d = jax.lax.dot_general(s0, xb, (((0,), (0,)), ((), ())),
                               preferred_element_type=f32)
    p0 = p_ref[...][:, 0:1]
    em0 = jnp.where(p0 > 0.5, em0d[:, d:], em0d[:, :d])
    gram = jax.lax.dot(mpw, mpw.T, preferred_element_type=f32)
    q = jax.lax.dot(em0.astype(bf16), gram.astype(bf16),
                    preferred_element_type=f32)
    qdup = jnp.concatenate([q, q], axis=1).astype(bf16)
    qbd = jax.lax.dot(sg, qdup,
                      preferred_element_type=f32).astype(bf16)
    prod = xb * qbd
    eye2 = jnp.concatenate(
        [jnp.concatenate([jnp.ones((d, 1), bf16), jnp.zeros((d, 1), bf16)],
                         axis=1),
         jnp.concatenate([jnp.zeros((d, 1), bf16), jnp.ones((d, 1), bf16)],
                         axis=1)], axis=0)
    dots2 = jax.lax.dot(prod, eye2, preferred_element_type=f32)
    dots1 = jnp.where(pf > 0.5, dots2[:, 1:2], dots2[:, 0:1])
    dots = dots1.reshape(bb, ll)

    lidx = jax.lax.broadcasted_iota(jnp.int32, (bb, ll), 1)
    mm = jnp.where(lidx > 0, m_ref[...], 0.0)
    num = jnp.sum(alpha * dots * mm, axis=1, keepdims=True) * alpha[:, 0:1]
    den = jnp.sum(mm, axis=1, keepdims=True) + 1e-8
    out_ref[...] = num / den


def _tc_fused(x128, par2, mask, mp_w, W_a, v_a):
    b, ll = par2.shape
    d = mp_w.shape[0]
    bb = 128
    flat = bb * ll

    r = jnp.arange(flat, dtype=jnp.int32)
    sg = (r[:, None] // ll == jnp.arange(bb)[None, :]).astype(jnp.bfloat16)
    s0 = (r[:, None] == (ll * jnp.arange(bb))[None, :]).astype(jnp.bfloat16)
    m1 = (r[:, None] % ll == jnp.arange(ll)[None, :]).astype(jnp.bfloat16)

    return pl.pallas_call(
        _fused_body,
        grid=(b // bb,),
        in_specs=[
            pl.BlockSpec((flat, 2 * d), lambda i: (i, 0)),
            pl.BlockSpec((bb, ll), lambda i: (i, 0)),
            pl.BlockSpec((bb, ll), lambda i: (i, 0)),
            pl.BlockSpec((flat, bb), lambda i: (0, 0)),
            pl.BlockSpec((flat, bb), lambda i: (0, 0)),
            pl.BlockSpec((flat, ll), lambda i: (0, 0)),
            pl.BlockSpec((d, d), lambda i: (0, 0)),
            pl.BlockSpec((d, W_a.shape[1]), lambda i: (0, 0)),
            pl.BlockSpec((W_a.shape[1], 1), lambda i: (0, 0)),
        ],
        out_specs=pl.BlockSpec((bb, 1), lambda i: (i, 0)),
        out_shape=jax.ShapeDtypeStruct((b, 1), jnp.float32),
    )(x128, par2, mask, sg, s0, m1, mp_w, W_a, v_a.reshape(-1, 1))


def kernel(inds, mask, table, mp_w, W_a, v_a):
    b, ll = inds.shape
    v, d = table.shape
    inds32 = inds.astype(jnp.int32)
    table2 = _tc_pair_table(table.T)
    x128 = _sc_gather(table2, (inds32 >> 1).reshape(-1))
    par2 = (inds32 & 1).astype(jnp.float32)
    scores = _tc_fused(x128, par2, mask, mp_w, W_a, v_a)
    return scores.reshape(b)


# speedup vs baseline: 1.2813x; 1.2813x over previous
import functools

import jax
import jax.numpy as jnp
from jax.experimental import pallas as pl
from jax.experimental.pallas import tpu as pltpu
from jax.experimental.pallas import tpu_sc as plsc


def _dup_body(t_ref, out_ref):
    x = t_ref[...]
    d = x.shape[0]
    ri = jax.lax.broadcasted_iota(jnp.int32, (d, 2 * d), 0)
    ci = jax.lax.broadcasted_iota(jnp.int32, (d, 2 * d), 1)
    sel = ((ri == ci) | (ri == ci - d)).astype(jnp.float32)
    out_ref[...] = jax.lax.dot_general(
        x, sel, (((0,), (0,)), ((), ())), preferred_element_type=jnp.float32)


def _tc_dup_table(tt):
    d, v = tt.shape
    w = 12800
    grid = (v + w - 1) // w
    return pl.pallas_call(
        _dup_body,
        grid=(grid,),
        in_specs=[pl.BlockSpec((d, w), lambda i: (0, i))],
        out_specs=pl.BlockSpec((w, 2 * d), lambda i: (i, 0)),
        out_shape=jax.ShapeDtypeStruct((v, 2 * d), jnp.float32),
    )(tt)


def _sc_gather(table2, idx):
    n = idx.shape[0]
    d = table2.shape[1]
    nc, ns = 2, 16
    nw = nc * ns
    b_per_w = n // nw
    ch = 800
    n_ch = b_per_w // ch
    mesh = plsc.VectorSubcoreMesh(core_axis_name="c", subcore_axis_name="s")

    @functools.partial(
        pl.kernel, mesh=mesh,
        out_type=jax.ShapeDtypeStruct((n, d), jnp.float32),
        scratch_types=[
            pltpu.VMEM((ch,), jnp.int32),
            pltpu.VMEM((ch, d), jnp.float32),
            pltpu.SemaphoreType.DMA,
        ],
    )
    def k(table_hbm, idx_hbm, out_hbm, idx_v, rows_v, sem):
        wid = jax.lax.axis_index("s") * nc + jax.lax.axis_index("c")
        base = wid * b_per_w

        @pl.loop(0, n_ch)
        def _(c):
            off = base + c * ch
            pltpu.sync_copy(idx_hbm.at[pl.ds(off, ch)], idx_v)
            pltpu.async_copy(table_hbm.at[idx_v], rows_v, sem).wait()
            pltpu.sync_copy(rows_v, out_hbm.at[pl.ds(off, ch)])

    return k(table2, idx)


def _fused_body(x_ref, m_ref, mpw_ref, wa_ref, va_ref, out_ref):
    bb, ll = m_ref.shape
    d = x_ref.shape[1] // 2

    x = x_ref[...]
    em = x[:, :d]

    mpw = mpw_ref[...]
    wc = jax.lax.dot(mpw, wa_ref[...], preferred_element_type=jnp.float32)
    t = jnp.tanh(jax.lax.dot(em, wc, preferred_element_type=jnp.float32))
    g1 = jax.lax.dot(t, va_ref[...], preferred_element_type=jnp.float32)
    g = g1.reshape(bb, ll)

    mx = jnp.max(g, axis=1, keepdims=True)
    e = jnp.exp(g - mx)
    alpha = e / jnp.sum(e, axis=1, keepdims=True)

    em3 = em.reshape(bb, ll, d)
    em0 = em3[:, 0, :]
    gram = jax.lax.dot(mpw, mpw.T, preferred_element_type=jnp.float32)
    q = jax.lax.dot(em0, gram, preferred_element_type=jnp.float32)
    dots = jnp.sum(em3 * q[:, None, :], axis=2)

    lidx = jax.lax.broadcasted_iota(jnp.int32, (bb, ll), 1)
    mm = jnp.where(lidx > 0, m_ref[...], 0.0)
    num = jnp.sum(alpha * dots * mm, axis=1, keepdims=True) * alpha[:, 0:1]
    den = jnp.sum(mm, axis=1, keepdims=True) + 1e-8
    out_ref[...] = num / den


def _tc_fused(x128, mask, mp_w, W_a, v_a):
    b, ll = mask.shape
    d = mp_w.shape[0]
    bb = 128
    flat = bb * ll
    return pl.pallas_call(
        _fused_body,
        grid=(b // bb,),
        in_specs=[
            pl.BlockSpec((flat, 2 * d), lambda i: (i, 0)),
            pl.BlockSpec((bb, ll), lambda i: (i, 0)),
            pl.BlockSpec((d, d), lambda i: (0, 0)),
            pl.BlockSpec((d, W_a.shape[1]), lambda i: (0, 0)),
            pl.BlockSpec((W_a.shape[1], 1), lambda i: (0, 0)),
        ],
        out_specs=pl.BlockSpec((bb, 1), lambda i: (i, 0)),
        out_shape=jax.ShapeDtypeStruct((b, 1), jnp.float32),
    )(x128, mask, mp_w, W_a, v_a.reshape(-1, 1))


def kernel(inds, mask, table, mp_w, W_a, v_a):
    b, ll = inds.shape
    v, d = table.shape
    inds32 = inds.astype(jnp.int32)
    table2 = _tc_dup_table(table.T)
    x128 = _sc_gather(table2, inds32.reshape(-1))
    scores = _tc_fused(x128, mask, mp_w, W_a, v_a)
    return scores.reshape(b)


# speedup vs baseline: 1.2900x; 1.0068x over previous
import functools

import jax
import jax.numpy as jnp
from jax.experimental import pallas as pl
from jax.experimental.pallas import tpu as pltpu
from jax.experimental.pallas import tpu_sc as plsc


def _dup_body(t_ref, out_ref):
    x = t_ref[...]
    d = x.shape[0]
    ri = jax.lax.broadcasted_iota(jnp.int32, (d, 2 * d), 0)
    ci = jax.lax.broadcasted_iota(jnp.int32, (d, 2 * d), 1)
    sel = ((ri == ci) | (ri == ci - d)).astype(jnp.float32)
    out_ref[...] = jax.lax.dot_general(
        x, sel, (((0,), (0,)), ((), ())), preferred_element_type=jnp.float32)


def _tc_dup_table(tt):
    d, v = tt.shape
    w = 12800
    grid = (v + w - 1) // w
    return pl.pallas_call(
        _dup_body,
        grid=(grid,),
        in_specs=[pl.BlockSpec((d, w), lambda i: (0, i))],
        out_specs=pl.BlockSpec((w, 2 * d), lambda i: (i, 0)),
        out_shape=jax.ShapeDtypeStruct((v, 2 * d), jnp.float32),
    )(tt)


def _sc_gather(table2, idx):
    n = idx.shape[0]
    d = table2.shape[1]
    nc, ns = 2, 16
    nw = nc * ns
    b_per_w = n // nw
    ch = 800
    n_ch = b_per_w // ch
    mesh = plsc.VectorSubcoreMesh(core_axis_name="c", subcore_axis_name="s")

    @functools.partial(
        pl.kernel, mesh=mesh,
        out_type=jax.ShapeDtypeStruct((n, d), jnp.float32),
        scratch_types=[
            pltpu.VMEM((ch,), jnp.int32),
            pltpu.VMEM((ch, d), jnp.float32),
            pltpu.SemaphoreType.DMA,
        ],
    )
    def k(table_hbm, idx_hbm, out_hbm, idx_v, rows_v, sem):
        wid = jax.lax.axis_index("s") * nc + jax.lax.axis_index("c")
        base = wid * b_per_w

        @pl.loop(0, n_ch)
        def _(c):
            off = base + c * ch
            pltpu.sync_copy(idx_hbm.at[pl.ds(off, ch)], idx_v)
            pltpu.async_copy(table_hbm.at[idx_v], rows_v, sem).wait()
            pltpu.sync_copy(rows_v, out_hbm.at[pl.ds(off, ch)])

    return k(table2, idx)


def _fused_body(x_ref, m_ref, mpw_ref, wa_ref, va_ref, out_ref):
    bb, ll = m_ref.shape
    d = x_ref.shape[1] // 2

    x = x_ref[...]

    mpw = mpw_ref[...]
    wc = jax.lax.dot(mpw, wa_ref[...], preferred_element_type=jnp.float32)
    wc2 = jnp.concatenate([wc, jnp.zeros((d, d), jnp.float32)], axis=0)
    t = jnp.tanh(jax.lax.dot(x, wc2, preferred_element_type=jnp.float32))
    g1 = jax.lax.dot(t, va_ref[...], preferred_element_type=jnp.float32)
    g = g1.reshape(bb, ll)

    mx = jnp.max(g, axis=1, keepdims=True)
    e = jnp.exp(g - mx)
    alpha = e / jnp.sum(e, axis=1, keepdims=True)

    x3 = x.reshape(bb, ll, 2 * d)
    em0 = x3[:, 0, :][:, :d]
    gram = jax.lax.dot(mpw, mpw.T, preferred_element_type=jnp.float32)
    q = jax.lax.dot(em0, gram, preferred_element_type=jnp.float32)
    qpad = jnp.concatenate([q, jnp.zeros_like(q)], axis=1)
    dots = jnp.sum(x3 * qpad[:, None, :], axis=2)

    lidx = jax.lax.broadcasted_iota(jnp.int32, (bb, ll), 1)
    mm = jnp.where(lidx > 0, m_ref[...], 0.0)
    num = jnp.sum(alpha * dots * mm, axis=1, keepdims=True) * alpha[:, 0:1]
    den = jnp.sum(mm, axis=1, keepdims=True) + 1e-8
    out_ref[...] = num / den


def _tc_fused(x128, mask, mp_w, W_a, v_a):
    b, ll = mask.shape
    d = mp_w.shape[0]
    bb = 128
    flat = bb * ll
    return pl.pallas_call(
        _fused_body,
        grid=(b // bb,),
        in_specs=[
            pl.BlockSpec((flat, 2 * d), lambda i: (i, 0)),
            pl.BlockSpec((bb, ll), lambda i: (i, 0)),
            pl.BlockSpec((d, d), lambda i: (0, 0)),
            pl.BlockSpec((d, W_a.shape[1]), lambda i: (0, 0)),
            pl.BlockSpec((W_a.shape[1], 1), lambda i: (0, 0)),
        ],
        out_specs=pl.BlockSpec((bb, 1), lambda i: (i, 0)),
        out_shape=jax.ShapeDtypeStruct((b, 1), jnp.float32),
    )(x128, mask, mp_w, W_a, v_a.reshape(-1, 1))


def kernel(inds, mask, table, mp_w, W_a, v_a):
    b, ll = inds.shape
    v, d = table.shape
    inds32 = inds.astype(jnp.int32)
    table2 = _tc_dup_table(table.T)
    x128 = _sc_gather(table2, inds32.reshape(-1))
    scores = _tc_fused(x128, mask, mp_w, W_a, v_a)
    return scores.reshape(b)
